# baseline (speedup 1.0000x reference)
import functools

import jax
import jax.numpy as jnp
from jax import lax
from jax.experimental import pallas as pl
from jax.experimental.pallas import tpu as pltpu
from jax.experimental.pallas import tpu_sc as plsc

N = 10000
NP = 10240
E = 320000
NC = 2
NS = 16
NW = NC * NS
CH = 128
NCH = 80
EPWP = NCH * CH
EPAD = NW * EPWP - E
PADV = 10200
RPS = NP // NS
RB = 2048
G = NP // RB


def _mesh():
  return plsc.VectorSubcoreMesh(
      core_axis_name="c", subcore_axis_name="s", num_cores=NC, num_subcores=NS)


def _sc_degrees(idxr, ones_h, zeros16):
  @functools.partial(
      pl.kernel,
      out_type=jax.ShapeDtypeStruct((NC, 2, NP, 16), jnp.float32),
      mesh=_mesh(),
      compiler_params=pltpu.CompilerParams(use_tc_tiling_on_sc=False),
      scratch_types=[
          pltpu.VMEM((2 * NCH, CH), jnp.int32),
          pltpu.VMEM((CH, 16), jnp.float32),
          pltpu.VMEM_SHARED((NP, 16), jnp.float32),
          pltpu.VMEM_SHARED((NP, 16), jnp.float32),
      ],
  )
  def k(idx_hbm, ones_hbm, zeros_hbm, out_hbm, idx_v, ones_v, deg_s, deg_d):
    cid = lax.axis_index("c")
    sid = lax.axis_index("s")
    w = cid * NS + sid
    r0 = sid * RPS
    pltpu.sync_copy(zeros_hbm.at[pl.ds(r0, RPS)], deg_s.at[pl.ds(r0, RPS)])
    pltpu.sync_copy(zeros_hbm.at[pl.ds(r0, RPS)], deg_d.at[pl.ds(r0, RPS)])
    pltpu.sync_copy(ones_hbm, ones_v)
    pltpu.sync_copy(idx_hbm.at[w], idx_v)
    plsc.subcore_barrier()

    def step(j, c):
      pltpu.sync_copy(ones_v, deg_s.at[idx_v.at[2 * j]], add=True)
      pltpu.sync_copy(ones_v, deg_d.at[idx_v.at[2 * j + 1]], add=True)
      return c

    lax.fori_loop(0, NCH, step, 0)
    plsc.subcore_barrier()
    pltpu.sync_copy(deg_s.at[pl.ds(r0, RPS)],
                    out_hbm.at[cid, 0, pl.ds(r0, RPS)])
    pltpu.sync_copy(deg_d.at[pl.ds(r0, RPS)],
                    out_hbm.at[cid, 1, pl.ds(r0, RPS)])

  return k(idxr, ones_h, zeros16)


def _sc_aggregate(xs, idx2, zeros_f, feat):
  @functools.partial(
      pl.kernel,
      out_type=jax.ShapeDtypeStruct((NC, NP, feat), jnp.float32),
      mesh=_mesh(),
      compiler_params=(None if feat % 128 == 0 else
                       pltpu.CompilerParams(use_tc_tiling_on_sc=False)),
      scratch_types=[
          pltpu.VMEM((2, CH), jnp.int32),
          pltpu.VMEM((2, CH), jnp.int32),
          pltpu.VMEM((CH, feat), jnp.float32),
          pltpu.VMEM((CH, feat), jnp.float32),
          pltpu.SemaphoreType.DMA,
          pltpu.SemaphoreType.DMA,
          pltpu.SemaphoreType.DMA,
          pltpu.SemaphoreType.DMA,
          pltpu.VMEM_SHARED((NP, feat), jnp.float32),
      ],
  )
  def k(xs_hbm, idx_hbm, zeros_hbm, out_hbm,
        ib0, ib1, rows0, rows1, sg0, sg1, si0, si1, acc):
    cid = lax.axis_index("c")
    sid = lax.axis_index("s")
    w = cid * NS + sid
    r0 = sid * RPS
    pltpu.sync_copy(zeros_hbm.at[pl.ds(r0, RPS)], acc.at[pl.ds(r0, RPS)])
    pltpu.sync_copy(idx_hbm.at[w, 0], ib0)
    plsc.subcore_barrier()
    pltpu.async_copy(xs_hbm.at[ib0.at[0]], rows0, sg0)
    pltpu.async_copy(idx_hbm.at[w, 1], ib1, si1)

    def step(i, c):
      j = 2 * i
      pltpu.make_async_copy(xs_hbm.at[ib0.at[0]], rows0, sg0).wait()
      pltpu.make_async_copy(idx_hbm.at[w, 0], ib1, si1).wait()
      pltpu.async_copy(xs_hbm.at[ib1.at[0]], rows1, sg1)
      pltpu.sync_copy(rows0, acc.at[ib0.at[1]], add=True)
      j2 = jnp.where(j + 2 < NCH, j + 2, 0)
      pltpu.async_copy(idx_hbm.at[w, j2], ib0, si0)
      pltpu.make_async_copy(xs_hbm.at[ib1.at[0]], rows1, sg1).wait()
      pltpu.make_async_copy(idx_hbm.at[w, 0], ib0, si0).wait()
      pltpu.async_copy(xs_hbm.at[ib0.at[0]], rows0, sg0)
      pltpu.sync_copy(rows1, acc.at[ib1.at[1]], add=True)
      j3 = jnp.where(j + 3 < NCH, j + 3, 0)
      pltpu.async_copy(idx_hbm.at[w, j3], ib1, si1)
      return c

    lax.fori_loop(0, NCH // 2, step, 0)
    pltpu.make_async_copy(xs_hbm.at[ib0.at[0]], rows0, sg0).wait()
    pltpu.make_async_copy(idx_hbm.at[w, 0], ib1, si1).wait()
    plsc.subcore_barrier()
    pltpu.sync_copy(acc.at[pl.ds(r0, RPS)], out_hbm.at[cid, pl.ds(r0, RPS)])

  return k(xs, idx2, zeros_f)


def _tc_prescale(parts, features):
  def body(p_ref, x_ref, xs_ref, nsb_ref, ndb_ref):
    outdeg = p_ref[0, 0, :, 0:1] + p_ref[1, 0, :, 0:1]
    indeg = p_ref[0, 1, :, 0:1] + p_ref[1, 1, :, 0:1]
    nsrc = lax.rsqrt(jnp.maximum(outdeg, 1.0))
    ndst = lax.rsqrt(jnp.maximum(indeg, 1.0))
    xs_ref[...] = x_ref[...] * nsrc
    nsb_ref[...] = jnp.broadcast_to(nsrc, (RB, 128))
    ndb_ref[...] = jnp.broadcast_to(ndst, (RB, 128))

  return pl.pallas_call(
      body,
      grid=(G,),
      in_specs=[
          pl.BlockSpec((NC, 2, RB, 16), lambda i: (0, 0, i, 0)),
          pl.BlockSpec((RB, 128), lambda i: (i, 0)),
      ],
      out_specs=[pl.BlockSpec((RB, 128), lambda i: (i, 0))] * 3,
      out_shape=[jax.ShapeDtypeStruct((NP, 128), jnp.float32)] * 3,
  )(parts, features)


def _tc_layer1(agg, ndb, nsb, W, b):
  def body(a_ref, ndb_ref, nsb_ref, w_ref, b_ref, o_ref):
    a = (a_ref[0] + a_ref[1]) * ndb_ref[...]
    h = jnp.dot(a, w_ref[...], preferred_element_type=jnp.float32)
    o_ref[...] = jnp.maximum(h + b_ref[...], 0.0) * nsb_ref[...]

  return pl.pallas_call(
      body,
      grid=(G,),
      in_specs=[
          pl.BlockSpec((NC, RB, 128), lambda i: (0, i, 0)),
          pl.BlockSpec((RB, 128), lambda i: (i, 0)),
          pl.BlockSpec((RB, 128), lambda i: (i, 0)),
          pl.BlockSpec((128, 128), lambda i: (0, 0)),
          pl.BlockSpec((1, 128), lambda i: (0, 0)),
      ],
      out_specs=pl.BlockSpec((RB, 128), lambda i: (i, 0)),
      out_shape=jax.ShapeDtypeStruct((NP, 128), jnp.float32),
  )(agg, ndb, nsb, W, b)


def _tc_layer2_proj(agg, ndb, nsb, W2, b2, W3p):
  def body(a_ref, ndb_ref, nsb_ref, w2_ref, b2_ref, w3_ref, o_ref):
    a = (a_ref[0] + a_ref[1]) * ndb_ref[...]
    h = jnp.dot(a, w2_ref[...], preferred_element_type=jnp.float32)
    h = jnp.maximum(h + b2_ref[...], 0.0)
    t = jnp.dot(h, w3_ref[...], preferred_element_type=jnp.float32)
    o_ref[...] = t * nsb_ref[...][:, :48]

  return pl.pallas_call(
      body,
      grid=(G,),
      in_specs=[
          pl.BlockSpec((NC, RB, 128), lambda i: (0, i, 0)),
          pl.BlockSpec((RB, 128), lambda i: (i, 0)),
          pl.BlockSpec((RB, 128), lambda i: (i, 0)),
          pl.BlockSpec((128, 128), lambda i: (0, 0)),
          pl.BlockSpec((1, 128), lambda i: (0, 0)),
          pl.BlockSpec((128, 48), lambda i: (0, 0)),
      ],
      out_specs=pl.BlockSpec((RB, 48), lambda i: (i, 0)),
      out_shape=jax.ShapeDtypeStruct((NP, 48), jnp.float32),
  )(agg, ndb, nsb, W2, b2, W3p)


def _tc_final(agg, ndb, b3p):
  def body(a_ref, ndb_ref, b_ref, o_ref):
    o_ref[...] = (a_ref[0] + a_ref[1]) * ndb_ref[...][:, :48] + b_ref[...]

  return pl.pallas_call(
      body,
      grid=(G,),
      in_specs=[
          pl.BlockSpec((NC, RB, 48), lambda i: (0, i, 0)),
          pl.BlockSpec((RB, 128), lambda i: (i, 0)),
          pl.BlockSpec((1, 48), lambda i: (0, 0)),
      ],
      out_specs=pl.BlockSpec((RB, 48), lambda i: (i, 0)),
      out_shape=jax.ShapeDtypeStruct((NP, 48), jnp.float32),
  )(agg, ndb, b3p)


def kernel(features, edge_index, W1, b1, W2, b2, W3, b3):
  src = edge_index[0].astype(jnp.int32)
  dst = edge_index[1].astype(jnp.int32)
  pad = jnp.full((EPAD,), PADV, jnp.int32)
  src4 = jnp.concatenate([src, pad]).reshape(NW, NCH, CH)
  dst4 = jnp.concatenate([dst, pad]).reshape(NW, NCH, CH)
  idx2 = jnp.stack([src4, dst4], axis=2)
  idxr = idx2.reshape(NW, 2 * NCH, CH)
  ones_h = jnp.ones((CH, 16), jnp.float32)
  z16 = jnp.zeros((NP, 16), jnp.float32)
  z128 = jnp.zeros((NP, 128), jnp.float32)
  z48 = jnp.zeros((NP, 48), jnp.float32)
  fpad = jnp.pad(features, ((0, NP - N), (0, 0)))

  parts = _sc_degrees(idxr, ones_h, z16)
  xs, nsb, ndb = _tc_prescale(parts, fpad)

  agg1 = _sc_aggregate(xs, idx2, z128, 128)
  h1s = _tc_layer1(agg1, ndb, nsb, W1, b1.reshape(1, 128))

  agg2 = _sc_aggregate(h1s, idx2, z128, 128)
  W3p = jnp.pad(W3, ((0, 0), (0, 8)))
  b3p = jnp.pad(b3, (0, 8)).reshape(1, 48)
  t3s = _tc_layer2_proj(agg2, ndb, nsb, W2, b2.reshape(1, 128), W3p)

  agg3 = _sc_aggregate(t3s, idx2, z48, 48)
  out48 = _tc_final(agg3, ndb, b3p)
  return out48[:N, :40]


# speedup vs baseline: 2.8462x; 2.8462x over previous
import functools

import jax
import jax.numpy as jnp
from jax import lax
from jax.experimental import pallas as pl
from jax.experimental.pallas import tpu as pltpu
from jax.experimental.pallas import tpu_sc as plsc

N = 10000
NP = 10240
E = 320000
NC = 2
NS = 16
NW = NC * NS
CH = 128
NCH = 80
EPWP = NCH * CH
EPAD = NW * EPWP - E
PADV = 10200
RPS = NP // NS
RB = 2048
G = NP // RB


def _mesh():
  return plsc.VectorSubcoreMesh(
      core_axis_name="c", subcore_axis_name="s", num_cores=NC, num_subcores=NS)


def _sc_degrees(idxr, ones_h, zeros16):
  @functools.partial(
      pl.kernel,
      out_type=jax.ShapeDtypeStruct((NC, 2, NP, 16), jnp.float32),
      mesh=_mesh(),
      compiler_params=pltpu.CompilerParams(use_tc_tiling_on_sc=False),
      scratch_types=[
          pltpu.VMEM((2 * NCH, CH), jnp.int32),
          pltpu.VMEM((CH, 16), jnp.float32),
          pltpu.VMEM_SHARED((NP, 16), jnp.float32),
          pltpu.VMEM_SHARED((NP, 16), jnp.float32),
      ],
  )
  def k(idx_hbm, ones_hbm, zeros_hbm, out_hbm, idx_v, ones_v, deg_s, deg_d):
    cid = lax.axis_index("c")
    sid = lax.axis_index("s")
    w = cid * NS + sid
    r0 = sid * RPS
    pltpu.sync_copy(zeros_hbm.at[pl.ds(r0, RPS)], deg_s.at[pl.ds(r0, RPS)])
    pltpu.sync_copy(zeros_hbm.at[pl.ds(r0, RPS)], deg_d.at[pl.ds(r0, RPS)])
    pltpu.sync_copy(ones_hbm, ones_v)
    pltpu.sync_copy(idx_hbm.at[w], idx_v)
    plsc.subcore_barrier()

    def step(j, c):
      pltpu.sync_copy(ones_v, deg_s.at[idx_v.at[2 * j]], add=True)
      pltpu.sync_copy(ones_v, deg_d.at[idx_v.at[2 * j + 1]], add=True)
      return c

    lax.fori_loop(0, NCH, step, 0)
    plsc.subcore_barrier()
    pltpu.sync_copy(deg_s.at[pl.ds(r0, RPS)],
                    out_hbm.at[cid, 0, pl.ds(r0, RPS)])
    pltpu.sync_copy(deg_d.at[pl.ds(r0, RPS)],
                    out_hbm.at[cid, 1, pl.ds(r0, RPS)])

  return k(idxr, ones_h, zeros16)


def _sc_aggregate(xs, idx2, zeros_f, feat):
  @functools.partial(
      pl.kernel,
      out_type=jax.ShapeDtypeStruct((NC, NP, feat), jnp.float32),
      mesh=_mesh(),
      compiler_params=(None if feat % 128 == 0 else
                       pltpu.CompilerParams(use_tc_tiling_on_sc=False)),
      scratch_types=[
          pltpu.VMEM((2, CH), jnp.int32),
          pltpu.VMEM((2, CH), jnp.int32),
          pltpu.VMEM((CH, feat), jnp.float32),
          pltpu.VMEM((CH, feat), jnp.float32),
          pltpu.SemaphoreType.DMA,
          pltpu.SemaphoreType.DMA,
          pltpu.SemaphoreType.DMA,
          pltpu.SemaphoreType.DMA,
          pltpu.VMEM_SHARED((NP, feat), jnp.float32),
      ],
  )
  def k(xs_hbm, idx_hbm, zeros_hbm, out_hbm,
        ib0, ib1, rows0, rows1, sg0, sg1, si0, si1, acc):
    cid = lax.axis_index("c")
    sid = lax.axis_index("s")
    w = cid * NS + sid
    r0 = sid * RPS
    pltpu.sync_copy(zeros_hbm.at[pl.ds(r0, RPS)], acc.at[pl.ds(r0, RPS)])
    pltpu.sync_copy(idx_hbm.at[w, 0], ib0)
    plsc.subcore_barrier()
    pltpu.async_copy(xs_hbm.at[ib0.at[0]], rows0, sg0)
    pltpu.async_copy(idx_hbm.at[w, 1], ib1, si1)

    def step(i, c):
      j = 2 * i
      pltpu.make_async_copy(xs_hbm.at[ib0.at[0]], rows0, sg0).wait()
      pltpu.make_async_copy(idx_hbm.at[w, 0], ib1, si1).wait()
      pltpu.async_copy(xs_hbm.at[ib1.at[0]], rows1, sg1)
      pltpu.sync_copy(rows0, acc.at[ib0.at[1]], add=True)
      j2 = jnp.where(j + 2 < NCH, j + 2, 0)
      pltpu.async_copy(idx_hbm.at[w, j2], ib0, si0)
      pltpu.make_async_copy(xs_hbm.at[ib1.at[0]], rows1, sg1).wait()
      pltpu.make_async_copy(idx_hbm.at[w, 0], ib0, si0).wait()
      pltpu.async_copy(xs_hbm.at[ib0.at[0]], rows0, sg0)
      pltpu.sync_copy(rows1, acc.at[ib1.at[1]], add=True)
      j3 = jnp.where(j + 3 < NCH, j + 3, 0)
      pltpu.async_copy(idx_hbm.at[w, j3], ib1, si1)
      return c

    lax.fori_loop(0, NCH // 2, step, 0)
    pltpu.make_async_copy(xs_hbm.at[ib0.at[0]], rows0, sg0).wait()
    pltpu.make_async_copy(idx_hbm.at[w, 0], ib1, si1).wait()
    plsc.subcore_barrier()
    pltpu.sync_copy(acc.at[pl.ds(r0, RPS)], out_hbm.at[cid, pl.ds(r0, RPS)])

  return k(xs, idx2, zeros_f)


def _tc_prescale(parts, features):
  def body(p_ref, x_ref, xs_ref, nsb_ref, ndb_ref):
    outdeg = p_ref[0, 0, :, 0:1] + p_ref[1, 0, :, 0:1]
    indeg = p_ref[0, 1, :, 0:1] + p_ref[1, 1, :, 0:1]
    nsrc = lax.rsqrt(jnp.maximum(outdeg, 1.0))
    ndst = lax.rsqrt(jnp.maximum(indeg, 1.0))
    xs_ref[...] = x_ref[...] * nsrc
    nsb_ref[...] = jnp.broadcast_to(nsrc, (RB, 128))
    ndb_ref[...] = jnp.broadcast_to(ndst, (RB, 128))

  return pl.pallas_call(
      body,
      grid=(G,),
      in_specs=[
          pl.BlockSpec((NC, 2, RB, 16), lambda i: (0, 0, i, 0)),
          pl.BlockSpec((RB, 128), lambda i: (i, 0)),
      ],
      out_specs=[pl.BlockSpec((RB, 128), lambda i: (i, 0))] * 3,
      out_shape=[jax.ShapeDtypeStruct((NP, 128), jnp.float32)] * 3,
  )(parts, features)


def _tc_layer1(agg, ndb, nsb, W, b):
  def body(a_ref, ndb_ref, nsb_ref, w_ref, b_ref, o_ref):
    a = (a_ref[0] + a_ref[1]) * ndb_ref[...]
    h = jnp.dot(a, w_ref[...], preferred_element_type=jnp.float32)
    o_ref[...] = jnp.maximum(h + b_ref[...], 0.0) * nsb_ref[...]

  return pl.pallas_call(
      body,
      grid=(G,),
      in_specs=[
          pl.BlockSpec((NC, RB, 128), lambda i: (0, i, 0)),
          pl.BlockSpec((RB, 128), lambda i: (i, 0)),
          pl.BlockSpec((RB, 128), lambda i: (i, 0)),
          pl.BlockSpec((128, 128), lambda i: (0, 0)),
          pl.BlockSpec((1, 128), lambda i: (0, 0)),
      ],
      out_specs=pl.BlockSpec((RB, 128), lambda i: (i, 0)),
      out_shape=jax.ShapeDtypeStruct((NP, 128), jnp.float32),
  )(agg, ndb, nsb, W, b)


def _tc_layer2_proj(agg, ndb, nsb, W2, b2, W3p):
  def body(a_ref, ndb_ref, nsb_ref, w2_ref, b2_ref, w3_ref, o_ref):
    a = (a_ref[0] + a_ref[1]) * ndb_ref[...]
    h = jnp.dot(a, w2_ref[...], preferred_element_type=jnp.float32)
    h = jnp.maximum(h + b2_ref[...], 0.0)
    t = jnp.dot(h, w3_ref[...], preferred_element_type=jnp.float32)
    o_ref[...] = t * nsb_ref[...][:, :48]

  return pl.pallas_call(
      body,
      grid=(G,),
      in_specs=[
          pl.BlockSpec((NC, RB, 128), lambda i: (0, i, 0)),
          pl.BlockSpec((RB, 128), lambda i: (i, 0)),
          pl.BlockSpec((RB, 128), lambda i: (i, 0)),
          pl.BlockSpec((128, 128), lambda i: (0, 0)),
          pl.BlockSpec((1, 128), lambda i: (0, 0)),
          pl.BlockSpec((128, 48), lambda i: (0, 0)),
      ],
      out_specs=pl.BlockSpec((RB, 48), lambda i: (i, 0)),
      out_shape=jax.ShapeDtypeStruct((NP, 48), jnp.float32),
  )(agg, ndb, nsb, W2, b2, W3p)


def _tc_final(agg, ndb, b3p):
  def body(a_ref, ndb_ref, b_ref, o_ref):
    o_ref[...] = (a_ref[0] + a_ref[1]) * ndb_ref[...][:, :48] + b_ref[...]

  return pl.pallas_call(
      body,
      grid=(G,),
      in_specs=[
          pl.BlockSpec((NC, RB, 48), lambda i: (0, i, 0)),
          pl.BlockSpec((RB, 128), lambda i: (i, 0)),
          pl.BlockSpec((1, 48), lambda i: (0, 0)),
      ],
      out_specs=pl.BlockSpec((RB, 48), lambda i: (i, 0)),
      out_shape=jax.ShapeDtypeStruct((NP, 48), jnp.float32),
  )(agg, ndb, b3p)


def kernel(features, edge_index, W1, b1, W2, b2, W3, b3):
  src = edge_index[0].astype(jnp.int32)
  dst = edge_index[1].astype(jnp.int32)
  pad = N + (jnp.arange(EPAD, dtype=jnp.int32) % (NP - N))
  src4 = jnp.concatenate([src, pad]).reshape(NW, NCH, CH)
  dst4 = jnp.concatenate([dst, pad]).reshape(NW, NCH, CH)
  idx2 = jnp.stack([src4, dst4], axis=2)
  idxr = idx2.reshape(NW, 2 * NCH, CH)
  ones_h = jnp.ones((CH, 16), jnp.float32)
  z16 = jnp.zeros((NP, 16), jnp.float32)
  z128 = jnp.zeros((NP, 128), jnp.float32)
  z48 = jnp.zeros((NP, 48), jnp.float32)
  fpad = jnp.pad(features, ((0, NP - N), (0, 0)))

  parts = _sc_degrees(idxr, ones_h, z16)
  xs, nsb, ndb = _tc_prescale(parts, fpad)

  agg1 = _sc_aggregate(xs, idx2, z128, 128)
  h1s = _tc_layer1(agg1, ndb, nsb, W1, b1.reshape(1, 128))

  agg2 = _sc_aggregate(h1s, idx2, z128, 128)
  W3p = jnp.pad(W3, ((0, 0), (0, 8)))
  b3p = jnp.pad(b3, (0, 8)).reshape(1, 48)
  t3s = _tc_layer2_proj(agg2, ndb, nsb, W2, b2.reshape(1, 128), W3p)

  agg3 = _sc_aggregate(t3s, idx2, z48, 48)
  out48 = _tc_final(agg3, ndb, b3p)
  return out48[:N, :40]
